# Initial kernel scaffold; baseline (speedup 1.0000x reference)
import jax
import jax.numpy as jnp
from jax.experimental import pallas as pl


def kernel(span_emb, mention_scores, W_fast, b_fast, emb_fast_distance, W_dist, b_dist, num_top_antecedents):
    raise NotImplementedError("write your pallas kernel here")


# speedup vs baseline: 2.4150x; 2.4150x over previous
import functools

import jax
import jax.numpy as jnp
from jax import lax
from jax.experimental import pallas as pl
from jax.experimental.pallas import tpu as pltpu

N = 2048
D = 2324
K = 50
KPAD = 64
BM = 256
NEG = -1e30


def _proj_kernel(span_ref, w_ref, b_ref, out_ref):
    acc = lax.dot_general(
        span_ref[...], w_ref[...],
        dimension_numbers=(((1,), (1,)), ((), ())),
        preferred_element_type=jnp.float32,
    )
    out_ref[...] = acc + b_ref[...]


def _score_topk_kernel(src_ref, span_ref, msr_ref, msc_ref, thr_ref, val_ref,
                       idx_out, vals_out, scores):
    i0 = pl.program_id(0) * BM
    dots = lax.dot_general(
        src_ref[...], span_ref[...],
        dimension_numbers=(((1,), (1,)), ((), ())),
        preferred_element_type=jnp.float32,
    )
    row = i0 + lax.broadcasted_iota(jnp.int32, (BM, N), 0)
    col = lax.broadcasted_iota(jnp.int32, (BM, N), 1)
    d = row - col
    bs = jnp.full((BM, N), val_ref[0], dtype=jnp.float32)
    for b in range(1, 10):
        bs = jnp.where(d >= thr_ref[b], val_ref[b], bs)
    score = msc_ref[...] + msr_ref[...]
    score = score + dots
    score = score + bs
    scores[...] = jnp.where(d >= 1, score, NEG)

    def body(k, carry):
        ov, oi = carry
        cur = scores[...]
        m = jnp.max(cur, axis=1, keepdims=True)
        hit = cur == m
        idx = jnp.min(jnp.where(hit, col, N), axis=1, keepdims=True)
        scores[...] = jnp.where(col == idx, -jnp.inf, cur)
        lane = lax.broadcasted_iota(jnp.int32, (BM, KPAD), 1)
        ov = jnp.where(lane == k, m, ov)
        oi = jnp.where(lane == k, idx, oi)
        return ov, oi

    ov, oi = lax.fori_loop(
        0, K, body,
        (jnp.zeros((BM, KPAD), jnp.float32), jnp.zeros((BM, KPAD), jnp.int32)),
    )
    idx_out[...] = oi
    vals_out[...] = ov


def _bucket_1d(dd):
    safe = jnp.maximum(dd, 1)
    logspace_idx = jnp.floor(
        jnp.log(safe.astype(jnp.float32)) / jnp.log(2.0)).astype(dd.dtype) + 3
    combined = jnp.where(dd <= 4, dd, logspace_idx)
    return jnp.clip(combined, 0, 9)


@functools.partial(jax.jit, static_argnames=())
def kernel(span_emb, mention_scores, W_fast, b_fast, emb_fast_distance,
           W_dist, b_dist, num_top_antecedents):
    del num_top_antecedents
    dist_vals = (emb_fast_distance @ W_dist.T + b_dist).reshape(10)
    b1d = _bucket_1d(jnp.arange(N, dtype=jnp.int32))
    thr = jnp.stack([jnp.argmax(b1d >= b).astype(jnp.int32) for b in range(10)])

    source = pl.pallas_call(
        _proj_kernel,
        grid=(N // BM,),
        in_specs=[
            pl.BlockSpec((BM, D), lambda i: (i, 0)),
            pl.BlockSpec((D, D), lambda i: (0, 0)),
            pl.BlockSpec((1, D), lambda i: (0, 0)),
        ],
        out_specs=pl.BlockSpec((BM, D), lambda i: (i, 0)),
        out_shape=jax.ShapeDtypeStruct((N, D), jnp.float32),
        compiler_params=pltpu.CompilerParams(
            dimension_semantics=("arbitrary",)),
    )(span_emb, W_fast, b_fast.reshape(1, D))

    idx, vals = pl.pallas_call(
        _score_topk_kernel,
        grid=(N // BM,),
        in_specs=[
            pl.BlockSpec((BM, D), lambda i: (i, 0)),
            pl.BlockSpec((N, D), lambda i: (0, 0)),
            pl.BlockSpec((1, N), lambda i: (0, 0)),
            pl.BlockSpec((BM, 1), lambda i: (i, 0)),
            pl.BlockSpec(memory_space=pltpu.SMEM),
            pl.BlockSpec(memory_space=pltpu.SMEM),
        ],
        out_specs=[
            pl.BlockSpec((BM, KPAD), lambda i: (i, 0)),
            pl.BlockSpec((BM, KPAD), lambda i: (i, 0)),
        ],
        out_shape=[
            jax.ShapeDtypeStruct((N, KPAD), jnp.int32),
            jax.ShapeDtypeStruct((N, KPAD), jnp.float32),
        ],
        scratch_shapes=[pltpu.VMEM((BM, N), jnp.float32)],
        compiler_params=pltpu.CompilerParams(
            dimension_semantics=("arbitrary",)),
    )(source, span_emb, mention_scores.reshape(1, N),
      mention_scores.reshape(N, 1), thr, dist_vals)

    top_idx = idx[:, :K]
    top_vals = vals[:, :K]
    rows = jnp.arange(N, dtype=jnp.int32)[:, None]
    mask = top_idx < rows
    offsets = rows - top_idx
    scores_out = jnp.where(mask, top_vals, -jnp.inf)
    return (top_idx, mask, scores_out, offsets)


# speedup vs baseline: 2.8670x; 1.1872x over previous
import jax
import jax.numpy as jnp
from jax import lax
from jax.experimental import pallas as pl
from jax.experimental.pallas import tpu as pltpu

N = 2048
D = 2324
K = 50
KPAD = 64
BM = 256
NEG = -1e30


def _proj_kernel(span_ref, w_ref, b_ref, out_ref):
    acc = lax.dot_general(
        span_ref[...], w_ref[...],
        dimension_numbers=(((1,), (1,)), ((), ())),
        preferred_element_type=jnp.float32,
    )
    out_ref[...] = acc + b_ref[...]


def _make_band_kernel(r0, wc):
    def body(src_ref, span_ref, msr_ref, msc_ref, thr_ref, val_ref,
             idx_out, vals_out, scores):
        i0 = r0 + pl.program_id(0) * BM
        dots = lax.dot_general(
            src_ref[...], span_ref[...],
            dimension_numbers=(((1,), (1,)), ((), ())),
            preferred_element_type=jnp.float32,
        )
        row = i0 + lax.broadcasted_iota(jnp.int32, (BM, wc), 0)
        col = lax.broadcasted_iota(jnp.int32, (BM, wc), 1)
        d = row - col
        bs = jnp.full((BM, wc), val_ref[0], dtype=jnp.float32)
        for b in range(1, 10):
            bs = jnp.where(d >= thr_ref[b], val_ref[b], bs)
        score = msc_ref[...] + msr_ref[...]
        score = score + dots
        score = score + bs
        scores[...] = jnp.where(d >= 1, score, NEG)

        def ext(k, carry):
            ov, oi = carry
            cur = scores[...]
            m = jnp.max(cur, axis=1, keepdims=True)
            hit = cur == m
            idx = jnp.min(jnp.where(hit, col, N), axis=1, keepdims=True)
            scores[...] = jnp.where(col == idx, -jnp.inf, cur)
            lane = lax.broadcasted_iota(jnp.int32, (BM, KPAD), 1)
            ov = jnp.where(lane == k, m, ov)
            oi = jnp.where(lane == k, idx, oi)
            return ov, oi

        ov, oi = lax.fori_loop(
            0, K, ext,
            (jnp.zeros((BM, KPAD), jnp.float32),
             jnp.zeros((BM, KPAD), jnp.int32)),
        )
        idx_out[...] = oi
        vals_out[...] = ov

    return body


def _band_call(source, span_emb, msr, msc, thr, dist_vals, r0, nrows, wc):
    return pl.pallas_call(
        _make_band_kernel(r0, wc),
        grid=(nrows // BM,),
        in_specs=[
            pl.BlockSpec((BM, D), lambda i: (r0 // BM + i, 0)),
            pl.BlockSpec((wc, D), lambda i: (0, 0)),
            pl.BlockSpec((1, wc), lambda i: (0, 0)),
            pl.BlockSpec((BM, 1), lambda i: (r0 // BM + i, 0)),
            pl.BlockSpec(memory_space=pltpu.SMEM),
            pl.BlockSpec(memory_space=pltpu.SMEM),
        ],
        out_specs=[
            pl.BlockSpec((BM, KPAD), lambda i: (i, 0)),
            pl.BlockSpec((BM, KPAD), lambda i: (i, 0)),
        ],
        out_shape=[
            jax.ShapeDtypeStruct((nrows, KPAD), jnp.int32),
            jax.ShapeDtypeStruct((nrows, KPAD), jnp.float32),
        ],
        scratch_shapes=[pltpu.VMEM((BM, wc), jnp.float32)],
        compiler_params=pltpu.CompilerParams(
            dimension_semantics=("arbitrary",)),
    )(source, span_emb, msr, msc, thr, dist_vals)


def _bucket_1d(dd):
    safe = jnp.maximum(dd, 1)
    logspace_idx = jnp.floor(
        jnp.log(safe.astype(jnp.float32)) / jnp.log(2.0)).astype(dd.dtype) + 3
    combined = jnp.where(dd <= 4, dd, logspace_idx)
    return jnp.clip(combined, 0, 9)


def kernel(span_emb, mention_scores, W_fast, b_fast, emb_fast_distance,
           W_dist, b_dist, num_top_antecedents):
    del num_top_antecedents
    dist_vals = (emb_fast_distance @ W_dist.T + b_dist).reshape(10)
    b1d = _bucket_1d(jnp.arange(N, dtype=jnp.int32))
    thr = jnp.stack([jnp.argmax(b1d >= b).astype(jnp.int32) for b in range(10)])

    source = pl.pallas_call(
        _proj_kernel,
        grid=(N // BM,),
        in_specs=[
            pl.BlockSpec((BM, D), lambda i: (i, 0)),
            pl.BlockSpec((D, D), lambda i: (0, 0)),
            pl.BlockSpec((1, D), lambda i: (0, 0)),
        ],
        out_specs=pl.BlockSpec((BM, D), lambda i: (i, 0)),
        out_shape=jax.ShapeDtypeStruct((N, D), jnp.float32),
        compiler_params=pltpu.CompilerParams(
            dimension_semantics=("arbitrary",)),
    )(span_emb, W_fast, b_fast.reshape(1, D))

    msr = mention_scores.reshape(1, N)
    msc = mention_scores.reshape(N, 1)
    idxs, vals = [], []
    for r0, nrows, wc in ((0, 512, 512), (512, 512, 1024),
                          (1024, 512, 1536), (1536, 512, 2048)):
        i_b, v_b = _band_call(source, span_emb, msr, msc, thr, dist_vals,
                              r0, nrows, wc)
        idxs.append(i_b)
        vals.append(v_b)
    idx = jnp.concatenate(idxs, axis=0)
    vals = jnp.concatenate(vals, axis=0)

    top_idx = idx[:, :K]
    top_vals = vals[:, :K]
    rows = jnp.arange(N, dtype=jnp.int32)[:, None]
    mask = top_idx < rows
    offsets = rows - top_idx
    scores_out = jnp.where(mask, top_vals, -jnp.inf)
    return (top_idx, mask, scores_out, offsets)


# speedup vs baseline: 2.9691x; 1.0356x over previous
import jax
import jax.numpy as jnp
from jax import lax
from jax.experimental import pallas as pl
from jax.experimental.pallas import tpu as pltpu

N = 2048
D = 2324
K = 50
KPAD = 64
BM = 256
NEG = -1e30


def _proj_kernel(span_ref, w_ref, b_ref, out_ref):
    acc = lax.dot_general(
        span_ref[...], w_ref[...],
        dimension_numbers=(((1,), (1,)), ((), ())),
        preferred_element_type=jnp.float32,
    )
    out_ref[...] = acc + b_ref[...]


def _score_topk_kernel(src_ref, span_ref, msr_ref, msc_ref, thr_ref, val_ref,
                       idx_out, vals_out, scores):
    pid = pl.program_id(0)
    i0 = pid * BM
    dots = lax.dot_general(
        src_ref[...], span_ref[...],
        dimension_numbers=(((1,), (1,)), ((), ())),
        preferred_element_type=jnp.float32,
    )
    row = i0 + lax.broadcasted_iota(jnp.int32, (BM, N), 0)
    col = lax.broadcasted_iota(jnp.int32, (BM, N), 1)
    d = row - col
    bs = jnp.full((BM, N), val_ref[0], dtype=jnp.float32)
    for b in range(1, 10):
        bs = jnp.where(d >= thr_ref[b], val_ref[b], bs)
    score = msc_ref[...] + msr_ref[...]
    score = score + dots
    score = score + bs
    scores[...] = jnp.where(d >= 1, score, NEG)

    def make_branch(wc):
        def branch():
            colw = lax.broadcasted_iota(jnp.int32, (BM, wc), 1)

            def ext(k, carry):
                ov, oi = carry
                cur = scores[:, :wc]
                m = jnp.max(cur, axis=1, keepdims=True)
                hit = cur == m
                idx = jnp.min(jnp.where(hit, colw, N), axis=1, keepdims=True)
                scores[:, :wc] = jnp.where(colw == idx, -jnp.inf, cur)
                lane = lax.broadcasted_iota(jnp.int32, (BM, KPAD), 1)
                ov = jnp.where(lane == k, m, ov)
                oi = jnp.where(lane == k, idx, oi)
                return ov, oi

            return lax.fori_loop(
                0, K, ext,
                (jnp.zeros((BM, KPAD), jnp.float32),
                 jnp.zeros((BM, KPAD), jnp.int32)),
            )
        return branch

    ov, oi = lax.switch(pid, [make_branch(BM * (b + 1)) for b in range(N // BM)])
    idx_out[...] = oi
    vals_out[...] = ov


def _bucket_1d(dd):
    safe = jnp.maximum(dd, 1)
    logspace_idx = jnp.floor(
        jnp.log(safe.astype(jnp.float32)) / jnp.log(2.0)).astype(dd.dtype) + 3
    combined = jnp.where(dd <= 4, dd, logspace_idx)
    return jnp.clip(combined, 0, 9)


def kernel(span_emb, mention_scores, W_fast, b_fast, emb_fast_distance,
           W_dist, b_dist, num_top_antecedents):
    del num_top_antecedents
    dist_vals = (emb_fast_distance @ W_dist.T + b_dist).reshape(10)
    b1d = _bucket_1d(jnp.arange(N, dtype=jnp.int32))
    thr = jnp.stack([jnp.argmax(b1d >= b).astype(jnp.int32) for b in range(10)])

    source = pl.pallas_call(
        _proj_kernel,
        grid=(N // BM,),
        in_specs=[
            pl.BlockSpec((BM, D), lambda i: (i, 0)),
            pl.BlockSpec((D, D), lambda i: (0, 0)),
            pl.BlockSpec((1, D), lambda i: (0, 0)),
        ],
        out_specs=pl.BlockSpec((BM, D), lambda i: (i, 0)),
        out_shape=jax.ShapeDtypeStruct((N, D), jnp.float32),
        compiler_params=pltpu.CompilerParams(
            dimension_semantics=("arbitrary",)),
    )(span_emb, W_fast, b_fast.reshape(1, D))

    idx, vals = pl.pallas_call(
        _score_topk_kernel,
        grid=(N // BM,),
        in_specs=[
            pl.BlockSpec((BM, D), lambda i: (i, 0)),
            pl.BlockSpec((N, D), lambda i: (0, 0)),
            pl.BlockSpec((1, N), lambda i: (0, 0)),
            pl.BlockSpec((BM, 1), lambda i: (i, 0)),
            pl.BlockSpec(memory_space=pltpu.SMEM),
            pl.BlockSpec(memory_space=pltpu.SMEM),
        ],
        out_specs=[
            pl.BlockSpec((BM, KPAD), lambda i: (i, 0)),
            pl.BlockSpec((BM, KPAD), lambda i: (i, 0)),
        ],
        out_shape=[
            jax.ShapeDtypeStruct((N, KPAD), jnp.int32),
            jax.ShapeDtypeStruct((N, KPAD), jnp.float32),
        ],
        scratch_shapes=[pltpu.VMEM((BM, N), jnp.float32)],
        compiler_params=pltpu.CompilerParams(
            dimension_semantics=("arbitrary",)),
    )(source, span_emb, mention_scores.reshape(1, N),
      mention_scores.reshape(N, 1), thr, dist_vals)

    top_idx = idx[:, :K]
    top_vals = vals[:, :K]
    rows = jnp.arange(N, dtype=jnp.int32)[:, None]
    mask = top_idx < rows
    offsets = rows - top_idx
    scores_out = jnp.where(mask, top_vals, -jnp.inf)
    return (top_idx, mask, scores_out, offsets)
